# baseline (speedup 1.0000x reference)
import functools

import jax
import jax.numpy as jnp
from jax import lax
from jax.experimental import pallas as pl
from jax.experimental.pallas import tpu as pltpu
from jax.experimental.pallas import tpu_sc as plsc

DIM = 32
NC = 2
NS = 16
NW = NC * NS
CHUNK = 1600
NSTREAM = 4


@functools.partial(jax.jit, static_argnums=(2,))
def _sc_gather(idx_flat, table, n):
    per_w = n // NW
    nchunk = per_w // CHUNK
    npair = nchunk // 2
    assert nchunk % 2 == 0 and nchunk >= 4
    mesh = plsc.VectorSubcoreMesh(core_axis_name="c", subcore_axis_name="s")

    @functools.partial(
        pl.kernel,
        out_type=jax.ShapeDtypeStruct((n, DIM), jnp.float32),
        mesh=mesh,
        scratch_types=[
            pltpu.VMEM((CHUNK,), jnp.int32),
            pltpu.VMEM((CHUNK,), jnp.int32),
            pltpu.VMEM((CHUNK, DIM), jnp.float32),
            pltpu.VMEM((CHUNK, DIM), jnp.float32),
            pltpu.SemaphoreType.DMA,
            pltpu.SemaphoreType.DMA,
        ],
        compiler_params=pltpu.CompilerParams(use_tc_tiling_on_sc=False),
    )
    def k(table_hbm, idx_hbm, out_hbm, idx0, idx1, rows0, rows1, g0, g1):
        wid = lax.axis_index("s") * NC + lax.axis_index("c")
        base = wid * per_w

        def idx_in(c, dst):
            pltpu.sync_copy(idx_hbm.at[pl.ds(base + c * CHUNK, CHUNK)], dst)

        def out_wr(c, src):
            pltpu.sync_copy(src, out_hbm.at[pl.ds(base + c * CHUNK, CHUNK)])

        sub = CHUNK // NSTREAM

        def fire(idx_v, rows_v, sem):
            for s in range(NSTREAM):
                pltpu.async_copy(
                    table_hbm.at[idx_v.at[pl.ds(s * sub, sub)]],
                    rows_v.at[pl.ds(s * sub, sub)],
                    sem,
                )

        def drain(idx_v, rows_v, sem):
            for s in range(NSTREAM):
                pltpu.make_async_copy(
                    table_hbm.at[idx_v.at[pl.ds(s * sub, sub)]],
                    rows_v.at[pl.ds(s * sub, sub)],
                    sem,
                ).wait()

        idx_in(0, idx0)
        fire(idx0, rows0, g0)

        @pl.loop(0, npair - 1)
        def _body(j):
            c = 2 * j
            idx_in(c + 1, idx1)
            drain(idx0, rows0, g0)
            fire(idx1, rows1, g1)
            out_wr(c, rows0)
            idx_in(c + 2, idx0)
            drain(idx1, rows1, g1)
            fire(idx0, rows0, g0)
            out_wr(c + 1, rows1)

        c = nchunk - 2
        idx_in(c + 1, idx1)
        drain(idx0, rows0, g0)
        fire(idx1, rows1, g1)
        out_wr(c, rows0)
        drain(idx1, rows1, g1)
        out_wr(c + 1, rows1)

    return k(table, idx_flat)


def kernel(indices, table):
    n = indices.size
    idx_flat = indices.reshape(-1).astype(jnp.int32)
    out = _sc_gather(idx_flat, table, n)
    return out.reshape(indices.shape + (DIM,))


# speedup vs baseline: 6.3804x; 6.3804x over previous
import functools

import jax
import jax.numpy as jnp
from jax import lax
from jax.experimental import pallas as pl
from jax.experimental.pallas import tpu as pltpu
from jax.experimental.pallas import tpu_sc as plsc

NC = 2
NS = 16
NW = NC * NS
BCH = 16


@functools.partial(jax.jit, static_argnums=(2, 3, 4))
def _sc_gather(idx_flat, table, b, l, d):
    lpad = -(-l // 8) * 8
    dpad = -(-d // 128) * 128
    per_w_b = b // NW
    chunk = BCH * l
    nchunk = per_w_b // BCH
    npair = nchunk // 2
    assert nchunk % 2 == 0 and nchunk >= 4
    mesh = plsc.VectorSubcoreMesh(core_axis_name="c", subcore_axis_name="s")

    @functools.partial(
        pl.kernel,
        out_type=jax.ShapeDtypeStruct((b, lpad, dpad), jnp.float32),
        mesh=mesh,
        scratch_types=[
            pltpu.VMEM((chunk,), jnp.int32),
            pltpu.VMEM((chunk,), jnp.int32),
            pltpu.VMEM((chunk, d), jnp.float32),
            pltpu.VMEM((chunk, d), jnp.float32),
            pltpu.SemaphoreType.DMA,
            pltpu.SemaphoreType.DMA,
            pltpu.SemaphoreType.DMA,
        ],
        compiler_params=pltpu.CompilerParams(use_tc_tiling_on_sc=False),
    )
    def k(table_hbm, idx_hbm, out_hbm, idx0, idx1, rows0, rows1, g0, g1, osem):
        wid = lax.axis_index("s") * NC + lax.axis_index("c")
        base = wid * (per_w_b * l)
        bbase = wid * per_w_b

        def idx_in(c, dst):
            pltpu.sync_copy(idx_hbm.at[pl.ds(base + c * chunk, chunk)], dst)

        def out_wr(c, src):
            b0 = bbase + c * BCH
            for j in range(BCH):
                pltpu.async_copy(
                    src.at[pl.ds(j * l, l)],
                    out_hbm.at[b0 + j, pl.ds(0, l), pl.ds(0, d)],
                    osem,
                )
            for j in range(BCH):
                pltpu.make_async_copy(
                    src.at[pl.ds(j * l, l)],
                    out_hbm.at[b0 + j, pl.ds(0, l), pl.ds(0, d)],
                    osem,
                ).wait()

        idx_in(0, idx0)
        pltpu.async_copy(table_hbm.at[idx0], rows0, g0)

        @pl.loop(0, npair - 1)
        def _body(p):
            c = 2 * p
            idx_in(c + 1, idx1)
            pltpu.make_async_copy(table_hbm.at[idx0], rows0, g0).wait()
            pltpu.async_copy(table_hbm.at[idx1], rows1, g1)
            out_wr(c, rows0)
            idx_in(c + 2, idx0)
            pltpu.make_async_copy(table_hbm.at[idx1], rows1, g1).wait()
            pltpu.async_copy(table_hbm.at[idx0], rows0, g0)
            out_wr(c + 1, rows1)

        c = nchunk - 2
        idx_in(c + 1, idx1)
        pltpu.make_async_copy(table_hbm.at[idx0], rows0, g0).wait()
        pltpu.async_copy(table_hbm.at[idx1], rows1, g1)
        out_wr(c, rows0)
        pltpu.make_async_copy(table_hbm.at[idx1], rows1, g1).wait()
        out_wr(c + 1, rows1)

    return k(table, idx_flat)


def kernel(indices, table):
    b, l = indices.shape
    d = table.shape[1]
    idx_flat = indices.reshape(-1).astype(jnp.int32)
    out = _sc_gather(idx_flat, table, b, l, d)
    return out[:, :l, :d]
